# baseline (speedup 1.0000x reference)
import functools

import jax
import jax.numpy as jnp
from jax import lax
from jax.experimental import pallas as pl
from jax.experimental.pallas import tpu as pltpu
from jax.experimental.pallas import tpu_sc as plsc

B = 16384
DM = 16
DL = 128
NROW = 100000
NPAD = 100352

_info = plsc.get_sparse_core_info()
_NC, _NS = _info.num_cores, _info.num_subcores
_NW = _NC * _NS
_BPW = B // _NW
_CH = 128
_NCH = _BPW // _CH
_SEG = 1568
_NSEG_G = _SEG // DM

_mesh = plsc.VectorSubcoreMesh(core_axis_name="c", subcore_axis_name="s")
_sc_params = pltpu.CompilerParams(use_tc_tiling_on_sc=False)


@functools.partial(
    pl.kernel,
    mesh=_mesh,
    compiler_params=_sc_params,
    out_type=[
        jax.ShapeDtypeStruct((B, DL), jnp.float32),
        jax.ShapeDtypeStruct((B, DL), jnp.float32),
    ],
    scratch_types=[
        pltpu.VMEM((_NCH, _CH), jnp.int32),
        pltpu.VMEM((_NCH, _CH), jnp.int32),
        pltpu.VMEM((_CH, DL), jnp.float32),
        pltpu.VMEM((_CH, DL), jnp.float32),
        pltpu.SemaphoreType.DMA,
    ],
)
def _sc_mlp_gather(user2, item2, u_mlp, i_mlp, xu_out, xi_out,
                   idx_u, idx_i, buf_a, buf_b, sem):
    wid = lax.axis_index("s") * _NC + lax.axis_index("c")
    base = wid * _BPW
    pltpu.sync_copy(user2.at[pl.ds(wid * _NCH, _NCH)], idx_u)
    pltpu.sync_copy(item2.at[pl.ds(wid * _NCH, _NCH)], idx_i)
    steps = [(u_mlp, idx_u, xu_out, c) for c in range(_NCH)]
    steps += [(i_mlp, idx_i, xi_out, c) for c in range(_NCH)]
    bufs = (buf_a, buf_b)
    prev = None
    for s, (tbl, idx, out, c) in enumerate(steps):
        cp = pltpu.async_copy(tbl.at[idx.at[c]], bufs[s % 2], sem)
        if prev is not None:
            p_cp, p_buf, p_out, p_c = prev
            p_cp.wait()
            pltpu.sync_copy(p_buf, p_out.at[pl.ds(base + p_c * _CH, _CH)])
        prev = (cp, bufs[s % 2], out, c)
    p_cp, p_buf, p_out, p_c = prev
    p_cp.wait()
    pltpu.sync_copy(p_buf, p_out.at[pl.ds(base + p_c * _CH, _CH)])


@functools.partial(
    pl.kernel,
    mesh=_mesh,
    compiler_params=pltpu.CompilerParams(use_tc_tiling_on_sc=False,
                                         needs_layout_passes=False),
    out_type=[
        jax.ShapeDtypeStruct((NPAD, DM), jnp.float32),
        jax.ShapeDtypeStruct((NPAD, DM), jnp.float32),
    ],
    scratch_types=[
        pltpu.VMEM((_NSEG_G * DM, DM), jnp.float32),
        pltpu.VMEM((_SEG, DM), jnp.float32),
        pltpu.SemaphoreType.DMA,
    ],
)
def _sc_detile(u3, i3, u_rows, i_rows, sbuf, obuf, sem):
    wid = lax.axis_index("s") * _NC + lax.axis_index("c")
    for tbl3, rows_out in ((u3, u_rows), (i3, i_rows)):
        for h in range(2):
            seg0 = wid * (2 * _NSEG_G) + h * _NSEG_G
            cps = [pltpu.async_copy(
                tbl3.at[k, pl.ds(seg0, _NSEG_G)],
                sbuf.at[pl.ds(k * _NSEG_G, _NSEG_G)], sem)
                for k in range(DM)]
            for cp in cps:
                cp.wait()

            def body(g, carry):
                rowv = lax.iota(jnp.int32, DM) + g * DM
                for k in range(DM):
                    v = sbuf[k * _NSEG_G + g, :]
                    plsc.store_scatter(
                        obuf, [rowv, jnp.full((DM,), k, jnp.int32)], v)
                return carry
            lax.fori_loop(0, _NSEG_G, body, 0)
            pltpu.sync_copy(
                obuf, rows_out.at[pl.ds(wid * 2 * _SEG + h * _SEG, _SEG)])


@functools.partial(
    pl.kernel,
    mesh=_mesh,
    compiler_params=pltpu.CompilerParams(use_tc_tiling_on_sc=False,
                                         needs_layout_passes=False),
    out_type=jax.ShapeDtypeStruct((B,), jnp.float32),
    scratch_types=[
        pltpu.VMEM((_NCH, _CH), jnp.int32),
        pltpu.VMEM((_NCH, _CH), jnp.int32),
        pltpu.VMEM((_BPW, DM), jnp.float32),
        pltpu.VMEM((_BPW, DM), jnp.float32),
        pltpu.VMEM((DM, DM), jnp.float32),
        pltpu.VMEM((_BPW,), jnp.float32),
        pltpu.SemaphoreType.DMA,
    ],
)
def _sc_mf(user2, item2, u_rows, i_rows, wp_mf, mfp_out,
           idx_u, idx_i, umf, imf, wpv, mfp, sem):
    wid = lax.axis_index("s") * _NC + lax.axis_index("c")
    base = wid * _BPW
    pltpu.sync_copy(user2.at[pl.ds(wid * _NCH, _NCH)], idx_u)
    pltpu.sync_copy(item2.at[pl.ds(wid * _NCH, _NCH)], idx_i)
    pltpu.sync_copy(wp_mf, wpv)
    cps = []
    for c in range(_NCH):
        cps.append(pltpu.async_copy(
            u_rows.at[idx_u.at[c]], umf.at[pl.ds(c * _CH, _CH)], sem))
        cps.append(pltpu.async_copy(
            i_rows.at[idx_i.at[c]], imf.at[pl.ds(c * _CH, _CH)], sem))
    for cp in cps:
        cp.wait()

    def body(g, carry):
        rowv = lax.iota(jnp.int32, DM) + g * DM
        acc = None
        for k in range(DM):
            vu = plsc.load_gather(umf, [rowv, jnp.full((DM,), k, jnp.int32)])
            vi = plsc.load_gather(imf, [rowv, jnp.full((DM,), k, jnp.int32)])
            t = vu * vi * wpv[k, :]
            acc = t if acc is None else acc + t
        plsc.store_scatter(mfp, [rowv], acc)
        return carry
    lax.fori_loop(0, _BPW // DM, body, 0)
    pltpu.sync_copy(mfp, mfp_out.at[pl.ds(base, _BPW)])


def _pack_body(in_ref, out_ref):
    for r in range(8):
        out_ref[pl.ds(r * NPAD, NROW)] = in_ref[r, :]
        out_ref[pl.ds(r * NPAD + NROW, NPAD - NROW)] = jnp.zeros(
            (NPAD - NROW,), jnp.float32)


def _pack_flat(table_t):
    return pl.pallas_call(
        _pack_body,
        grid=(2,),
        in_specs=[pl.BlockSpec((8, NROW), lambda i: (i, 0))],
        out_specs=pl.BlockSpec((8 * NPAD,), lambda i: (i,)),
        out_shape=jax.ShapeDtypeStruct((DM * NPAD,), jnp.float32),
    )(table_t)


_BLK = 4096


def _mlp_body(xu, xi, w0a, w0b, b0, w1, b1, w2, b2, w3, b3, wpx, bp, out):
    f32 = jnp.float32
    h = jnp.dot(xu[...], w0a[...], preferred_element_type=f32)
    h = h + jnp.dot(xi[...], w0b[...], preferred_element_type=f32)
    h = jnp.maximum(h + b0[...], 0.0)
    h = jnp.maximum(jnp.dot(h, w1[...], preferred_element_type=f32) + b1[...], 0.0)
    h = jnp.maximum(jnp.dot(h, w2[...], preferred_element_type=f32) + b2[...], 0.0)
    h = jnp.maximum(jnp.dot(h, w3[...], preferred_element_type=f32) + b3[...], 0.0)
    out[...] = jnp.sum(h * wpx[...], axis=1, keepdims=True) + bp[...]


def kernel(user, item, U_mf, I_mf, U_mlp, I_mlp,
           W0, b0, W1, b1, W2, b2, W3, b3, Wp, bp):
    user2 = user.astype(jnp.int32).reshape(_NW * _NCH, _CH)
    item2 = item.astype(jnp.int32).reshape(_NW * _NCH, _CH)
    xu, xi = _sc_mlp_gather(user2, item2, U_mlp, I_mlp)
    u3 = _pack_flat(U_mf.T).reshape(DM, NPAD // DM, DM)
    i3 = _pack_flat(I_mf.T).reshape(DM, NPAD // DM, DM)
    u_rows, i_rows = _sc_detile(u3, i3)
    wp_mf = jnp.broadcast_to(Wp[:DM].reshape(DM, 1), (DM, DM))
    mfp = _sc_mf(user2, item2, u_rows, i_rows, wp_mf)

    w0a, w0b = W0[:DL], W0[DL:]
    full = lambda shape: pl.BlockSpec(shape, lambda i: (0, 0))
    pred = pl.pallas_call(
        _mlp_body,
        grid=(B // _BLK,),
        in_specs=[
            pl.BlockSpec((_BLK, DL), lambda i: (i, 0)),
            pl.BlockSpec((_BLK, DL), lambda i: (i, 0)),
            full((DL, DL)), full((DL, DL)), full((1, DL)),
            full((DL, 64)), full((1, 64)),
            full((64, 32)), full((1, 32)),
            full((32, 16)), full((1, 16)),
            full((1, 16)), full((1, 1)),
        ],
        out_specs=pl.BlockSpec((_BLK, 1), lambda i: (i, 0)),
        out_shape=jax.ShapeDtypeStruct((B, 1), jnp.float32),
    )(xu, xi,
      w0a, w0b, b0.reshape(1, DL),
      W1, b1.reshape(1, 64),
      W2, b2.reshape(1, 32),
      W3, b3.reshape(1, 16),
      Wp[DM:].reshape(1, 16), bp.reshape(1, 1))
    return pred.reshape(-1) + mfp


# speedup vs baseline: 1.2371x; 1.2371x over previous
import functools

import jax
import jax.numpy as jnp
from jax import lax
from jax.experimental import pallas as pl
from jax.experimental.pallas import tpu as pltpu
from jax.experimental.pallas import tpu_sc as plsc

B = 16384
DM = 16
DL = 128
NROW = 100000
NPAD = 100096

_info = plsc.get_sparse_core_info()
_NC, _NS = _info.num_cores, _info.num_subcores
_NW = _NC * _NS
_BPW = B // _NW
_CH = 128
_NCH = _BPW // _CH

_mesh = plsc.VectorSubcoreMesh(core_axis_name="c", subcore_axis_name="s")
_sc_params = pltpu.CompilerParams(use_tc_tiling_on_sc=False)


@functools.partial(
    pl.kernel,
    mesh=_mesh,
    compiler_params=_sc_params,
    out_type=[
        jax.ShapeDtypeStruct((B, DL), jnp.float32),
        jax.ShapeDtypeStruct((B, DL), jnp.float32),
    ],
    scratch_types=[
        pltpu.VMEM((_NCH, _CH), jnp.int32),
        pltpu.VMEM((_NCH, _CH), jnp.int32),
        pltpu.VMEM((_CH, DL), jnp.float32),
        pltpu.VMEM((_CH, DL), jnp.float32),
        pltpu.SemaphoreType.DMA,
    ],
)
def _sc_mlp_gather(user2, item2, u_mlp, i_mlp, xu_out, xi_out,
                   idx_u, idx_i, buf_a, buf_b, sem):
    wid = lax.axis_index("s") * _NC + lax.axis_index("c")
    base = wid * _BPW
    pltpu.sync_copy(user2.at[pl.ds(wid * _NCH, _NCH)], idx_u)
    pltpu.sync_copy(item2.at[pl.ds(wid * _NCH, _NCH)], idx_i)
    steps = [(u_mlp, idx_u, xu_out, c) for c in range(_NCH)]
    steps += [(i_mlp, idx_i, xi_out, c) for c in range(_NCH)]
    bufs = (buf_a, buf_b)
    prev = None
    for s, (tbl, idx, out, c) in enumerate(steps):
        cp = pltpu.async_copy(tbl.at[idx.at[c]], bufs[s % 2], sem)
        if prev is not None:
            p_cp, p_buf, p_out, p_c = prev
            p_cp.wait()
            pltpu.sync_copy(p_buf, p_out.at[pl.ds(base + p_c * _CH, _CH)])
        prev = (cp, bufs[s % 2], out, c)
    p_cp, p_buf, p_out, p_c = prev
    p_cp.wait()
    pltpu.sync_copy(p_buf, p_out.at[pl.ds(base + p_c * _CH, _CH)])


@functools.partial(
    pl.kernel,
    mesh=_mesh,
    compiler_params=_sc_params,
    out_type=jax.ShapeDtypeStruct((B,), jnp.float32),
    scratch_types=[
        pltpu.VMEM((_NCH, _CH), jnp.int32),
        pltpu.VMEM((_NCH, _CH), jnp.int32),
        pltpu.VMEM((_NCH, DM, _CH), jnp.int32),
        pltpu.VMEM((_NCH, DM, _CH), jnp.int32),
        pltpu.VMEM((_NCH * DM, _CH), jnp.float32),
        pltpu.VMEM((_NCH * DM, _CH), jnp.float32),
        pltpu.VMEM((DM, DM), jnp.float32),
        pltpu.VMEM((_BPW,), jnp.float32),
        pltpu.SemaphoreType.DMA,
    ],
)
def _sc_mf(user2, item2, u_mf_flat, i_mf_flat, wp_mf, mfp_out,
           idx_u, idx_i, eidx_u, eidx_i, ucol, icol, wpv, mfp, sem):
    wid = lax.axis_index("s") * _NC + lax.axis_index("c")
    base = wid * _BPW
    pltpu.sync_copy(user2.at[pl.ds(wid * _NCH, _NCH)], idx_u)
    pltpu.sync_copy(item2.at[pl.ds(wid * _NCH, _NCH)], idx_i)
    pltpu.sync_copy(wp_mf, wpv)
    def _fire(c):
        for k in range(DM):
            for s in range(_CH // DM):
                sl = pl.ds(s * DM, DM)
                eidx_u[c, k, sl] = idx_u[c, sl] + (k * NPAD)
                eidx_i[c, k, sl] = idx_i[c, sl] + (k * NPAD)
        cps = []
        for k in range(DM):
            cps.append(pltpu.async_copy(
                u_mf_flat.at[eidx_u.at[c, k]], ucol.at[c * DM + k], sem))
            cps.append(pltpu.async_copy(
                i_mf_flat.at[eidx_i.at[c, k]], icol.at[c * DM + k], sem))
        return cps

    def _reduce(c):
        for s in range(_CH // DM):
            sl = pl.ds(s * DM, DM)
            acc = ucol[c * DM, sl] * icol[c * DM, sl] * wpv[0, :]
            for k in range(1, DM):
                acc = acc + ucol[c * DM + k, sl] * icol[c * DM + k, sl] * wpv[k, :]
            mfp[pl.ds(c * _CH + s * DM, DM)] = acc

    inflight = [_fire(c) for c in range(_NCH)]
    for c in range(_NCH):
        for cp in inflight[c]:
            cp.wait()
        _reduce(c)
    pltpu.sync_copy(mfp, mfp_out.at[pl.ds(base, _BPW)])


def _pack_body(in_ref, out_ref):
    for r in range(8):
        out_ref[pl.ds(r * NPAD, NROW)] = in_ref[r, :]
        out_ref[pl.ds(r * NPAD + NROW, NPAD - NROW)] = jnp.zeros(
            (NPAD - NROW,), jnp.float32)


def _pack_flat(table_t):
    return pl.pallas_call(
        _pack_body,
        grid=(2,),
        in_specs=[pl.BlockSpec((8, NROW), lambda i: (i, 0))],
        out_specs=pl.BlockSpec((8 * NPAD,), lambda i: (i,)),
        out_shape=jax.ShapeDtypeStruct((DM * NPAD,), jnp.float32),
    )(table_t)


_BLK = 4096


def _mlp_body(xu, xi, w0a, w0b, b0, w1, b1, w2, b2, w3, b3, wpx, bp, out):
    f32 = jnp.float32
    h = jnp.dot(xu[...], w0a[...], preferred_element_type=f32)
    h = h + jnp.dot(xi[...], w0b[...], preferred_element_type=f32)
    h = jnp.maximum(h + b0[...], 0.0)
    h = jnp.maximum(jnp.dot(h, w1[...], preferred_element_type=f32) + b1[...], 0.0)
    h = jnp.maximum(jnp.dot(h, w2[...], preferred_element_type=f32) + b2[...], 0.0)
    h = jnp.maximum(jnp.dot(h, w3[...], preferred_element_type=f32) + b3[...], 0.0)
    out[...] = jnp.sum(h * wpx[...], axis=1, keepdims=True) + bp[...]


def kernel(user, item, U_mf, I_mf, U_mlp, I_mlp,
           W0, b0, W1, b1, W2, b2, W3, b3, Wp, bp):
    user2 = user.astype(jnp.int32).reshape(_NW * _NCH, _CH)
    item2 = item.astype(jnp.int32).reshape(_NW * _NCH, _CH)
    xu, xi = _sc_mlp_gather(user2, item2, U_mlp, I_mlp)
    u_mf_flat = _pack_flat(U_mf.T)
    i_mf_flat = _pack_flat(I_mf.T)
    wp_mf = jnp.broadcast_to(Wp[:DM].reshape(DM, 1), (DM, DM))
    mfp = _sc_mf(user2, item2, u_mf_flat, i_mf_flat, wp_mf)

    w0a, w0b = W0[:DL], W0[DL:]
    full = lambda shape: pl.BlockSpec(shape, lambda i: (0, 0))
    pred = pl.pallas_call(
        _mlp_body,
        grid=(B // _BLK,),
        in_specs=[
            pl.BlockSpec((_BLK, DL), lambda i: (i, 0)),
            pl.BlockSpec((_BLK, DL), lambda i: (i, 0)),
            full((DL, DL)), full((DL, DL)), full((1, DL)),
            full((DL, 64)), full((1, 64)),
            full((64, 32)), full((1, 32)),
            full((32, 16)), full((1, 16)),
            full((1, 16)), full((1, 1)),
        ],
        out_specs=pl.BlockSpec((_BLK, 1), lambda i: (i, 0)),
        out_shape=jax.ShapeDtypeStruct((B, 1), jnp.float32),
    )(xu, xi,
      w0a, w0b, b0.reshape(1, DL),
      W1, b1.reshape(1, 64),
      W2, b2.reshape(1, 32),
      W3, b3.reshape(1, 16),
      Wp[DM:].reshape(1, 16), bp.reshape(1, 1))
    return pred.reshape(-1) + mfp


# speedup vs baseline: 1.2678x; 1.0248x over previous
import functools

import jax
import jax.numpy as jnp
from jax import lax
from jax.experimental import pallas as pl
from jax.experimental.pallas import tpu as pltpu
from jax.experimental.pallas import tpu_sc as plsc

B = 16384
DM = 16
DL = 128
NROW = 100000
NPAD = 100096

_info = plsc.get_sparse_core_info()
_NC, _NS = _info.num_cores, _info.num_subcores
_NW = _NC * _NS
_BPW = B // _NW
_CH = 128
_NCH = _BPW // _CH

_mesh = plsc.VectorSubcoreMesh(core_axis_name="c", subcore_axis_name="s")
_sc_params = pltpu.CompilerParams(use_tc_tiling_on_sc=False)


@functools.partial(
    pl.kernel,
    mesh=_mesh,
    compiler_params=_sc_params,
    out_type=[
        jax.ShapeDtypeStruct((B, DL), jnp.float32),
        jax.ShapeDtypeStruct((B, DL), jnp.float32),
    ],
    scratch_types=[
        pltpu.VMEM((_NCH, _CH), jnp.int32),
        pltpu.VMEM((_NCH, _CH), jnp.int32),
        pltpu.VMEM((_CH, DL), jnp.float32),
        pltpu.VMEM((_CH, DL), jnp.float32),
        pltpu.SemaphoreType.DMA,
    ],
)
def _sc_mlp_gather(user2, item2, u_mlp, i_mlp, xu_out, xi_out,
                   idx_u, idx_i, buf_a, buf_b, sem):
    wid = lax.axis_index("s") * _NC + lax.axis_index("c")
    base = wid * _BPW
    pltpu.sync_copy(user2.at[pl.ds(wid * _NCH, _NCH)], idx_u)
    pltpu.sync_copy(item2.at[pl.ds(wid * _NCH, _NCH)], idx_i)
    steps = [(u_mlp, idx_u, xu_out, c) for c in range(_NCH)]
    steps += [(i_mlp, idx_i, xi_out, c) for c in range(_NCH)]
    bufs = (buf_a, buf_b)
    prev = None
    for s, (tbl, idx, out, c) in enumerate(steps):
        cp = pltpu.async_copy(tbl.at[idx.at[c]], bufs[s % 2], sem)
        if prev is not None:
            p_cp, p_buf, p_out, p_c = prev
            p_cp.wait()
            pltpu.sync_copy(p_buf, p_out.at[pl.ds(base + p_c * _CH, _CH)])
        prev = (cp, bufs[s % 2], out, c)
    p_cp, p_buf, p_out, p_c = prev
    p_cp.wait()
    pltpu.sync_copy(p_buf, p_out.at[pl.ds(base + p_c * _CH, _CH)])


@functools.partial(
    pl.kernel,
    mesh=_mesh,
    compiler_params=_sc_params,
    out_type=jax.ShapeDtypeStruct((B,), jnp.float32),
    scratch_types=[
        pltpu.VMEM((_NCH, _CH), jnp.int32),
        pltpu.VMEM((_NCH, _CH), jnp.int32),
        pltpu.VMEM((2, DM, _CH), jnp.int32),
        pltpu.VMEM((2, DM, _CH), jnp.int32),
        pltpu.VMEM((DM, _CH), jnp.float32),
        pltpu.VMEM((DM, _CH), jnp.float32),
        pltpu.VMEM((DM, _CH), jnp.float32),
        pltpu.VMEM((DM, _CH), jnp.float32),
        pltpu.VMEM((DM, DM), jnp.float32),
        pltpu.VMEM((_BPW,), jnp.float32),
        pltpu.SemaphoreType.DMA,
    ],
)
def _sc_mf(user2, item2, u_mf_flat, i_mf_flat, wp_mf, mfp_out,
           idx_u, idx_i, eidx_u, eidx_i, ucol0, ucol1, icol0, icol1,
           wpv, mfp, sem):
    wid = lax.axis_index("s") * _NC + lax.axis_index("c")
    base = wid * _BPW
    pltpu.sync_copy(user2.at[pl.ds(wid * _NCH, _NCH)], idx_u)
    pltpu.sync_copy(item2.at[pl.ds(wid * _NCH, _NCH)], idx_i)
    pltpu.sync_copy(wp_mf, wpv)
    ubufs = (ucol0, ucol1)
    ibufs = (icol0, icol1)

    def _fire(c):
        eb = c % 2
        for k in range(DM):
            for s in range(_CH // DM):
                sl = pl.ds(s * DM, DM)
                eidx_u[eb, k, sl] = idx_u[c, sl] + (k * NPAD)
                eidx_i[eb, k, sl] = idx_i[c, sl] + (k * NPAD)
        cps = []
        for k in range(DM):
            cps.append(pltpu.async_copy(
                u_mf_flat.at[eidx_u.at[eb, k]], ubufs[eb].at[k], sem))
            cps.append(pltpu.async_copy(
                i_mf_flat.at[eidx_i.at[eb, k]], ibufs[eb].at[k], sem))
        return cps

    def _reduce(c):
        uc, ic = ubufs[c % 2], ibufs[c % 2]
        for s in range(_CH // DM):
            sl = pl.ds(s * DM, DM)
            acc = uc[0, sl] * ic[0, sl] * wpv[0, :]
            for k in range(1, DM):
                acc = acc + uc[k, sl] * ic[k, sl] * wpv[k, :]
            mfp[pl.ds(c * _CH + s * DM, DM)] = acc

    inflight = {0: _fire(0), 1: _fire(1)}
    for c in range(_NCH):
        for cp in inflight.pop(c):
            cp.wait()
        if c + 2 < _NCH:
            inflight[c + 2] = _fire(c + 2)
        _reduce(c)
    pltpu.sync_copy(mfp, mfp_out.at[pl.ds(base, _BPW)])


def _pack_body(in_ref, out_ref):
    for r in range(8):
        out_ref[pl.ds(r * NPAD, NROW)] = in_ref[r, :]
        out_ref[pl.ds(r * NPAD + NROW, NPAD - NROW)] = jnp.zeros(
            (NPAD - NROW,), jnp.float32)


def _pack_flat(table_t):
    return pl.pallas_call(
        _pack_body,
        grid=(2,),
        in_specs=[pl.BlockSpec((8, NROW), lambda i: (i, 0))],
        out_specs=pl.BlockSpec((8 * NPAD,), lambda i: (i,)),
        out_shape=jax.ShapeDtypeStruct((DM * NPAD,), jnp.float32),
    )(table_t)


_BLK = 4096


def _mlp_body(xu, xi, w0a, w0b, b0, w1, b1, w2, b2, w3, b3, wpx, bp, out):
    f32 = jnp.float32
    h = jnp.dot(xu[...], w0a[...], preferred_element_type=f32)
    h = h + jnp.dot(xi[...], w0b[...], preferred_element_type=f32)
    h = jnp.maximum(h + b0[...], 0.0)
    h = jnp.maximum(jnp.dot(h, w1[...], preferred_element_type=f32) + b1[...], 0.0)
    h = jnp.maximum(jnp.dot(h, w2[...], preferred_element_type=f32) + b2[...], 0.0)
    h = jnp.maximum(jnp.dot(h, w3[...], preferred_element_type=f32) + b3[...], 0.0)
    out[...] = jnp.sum(h * wpx[...], axis=1, keepdims=True) + bp[...]


def kernel(user, item, U_mf, I_mf, U_mlp, I_mlp,
           W0, b0, W1, b1, W2, b2, W3, b3, Wp, bp):
    user2 = user.astype(jnp.int32).reshape(_NW * _NCH, _CH)
    item2 = item.astype(jnp.int32).reshape(_NW * _NCH, _CH)
    xu, xi = _sc_mlp_gather(user2, item2, U_mlp, I_mlp)
    u_mf_flat = _pack_flat(U_mf.T)
    i_mf_flat = _pack_flat(I_mf.T)
    wp_mf = jnp.broadcast_to(Wp[:DM].reshape(DM, 1), (DM, DM))
    mfp = _sc_mf(user2, item2, u_mf_flat, i_mf_flat, wp_mf)

    w0a, w0b = W0[:DL], W0[DL:]
    full = lambda shape: pl.BlockSpec(shape, lambda i: (0, 0))
    pred = pl.pallas_call(
        _mlp_body,
        grid=(B // _BLK,),
        in_specs=[
            pl.BlockSpec((_BLK, DL), lambda i: (i, 0)),
            pl.BlockSpec((_BLK, DL), lambda i: (i, 0)),
            full((DL, DL)), full((DL, DL)), full((1, DL)),
            full((DL, 64)), full((1, 64)),
            full((64, 32)), full((1, 32)),
            full((32, 16)), full((1, 16)),
            full((1, 16)), full((1, 1)),
        ],
        out_specs=pl.BlockSpec((_BLK, 1), lambda i: (i, 0)),
        out_shape=jax.ShapeDtypeStruct((B, 1), jnp.float32),
    )(xu, xi,
      w0a, w0b, b0.reshape(1, DL),
      W1, b1.reshape(1, 64),
      W2, b2.reshape(1, 32),
      W3, b3.reshape(1, 16),
      Wp[DM:].reshape(1, 16), bp.reshape(1, 1))
    return pred.reshape(-1) + mfp
